# Initial kernel scaffold; baseline (speedup 1.0000x reference)
import jax
import jax.numpy as jnp
from jax.experimental import pallas as pl


def kernel(x, edge_index, W1, b1, W2, b2):
    raise NotImplementedError("write your pallas kernel here")


# speedup vs baseline: 20.2744x; 20.2744x over previous
import functools

import jax
import jax.numpy as jnp
from jax import lax
from jax.experimental import pallas as pl
from jax.experimental.pallas import tpu as pltpu
from jax.experimental.pallas import tpu_sc as plsc

N = 10000
NP = 10240
E = 320000
D = 128

NC = 2
NS = 16
NW = NC * NS
EPW = E // NW
K = 125
NCHUNK = EPW // K
RPT = NP // NS

_MESH = plsc.VectorSubcoreMesh(
    core_axis_name="c", subcore_axis_name="s", num_cores=NC, num_subcores=NS
)


def _deg_body(dst_hbm, ones_hbm, zeros_hbm, out_hbm, dst_v, ones_v, acc, sem):
    c = lax.axis_index("c")
    s = lax.axis_index("s")
    wid = c * NS + s
    pltpu.sync_copy(zeros_hbm.at[pl.ds(s * RPT, RPT)], acc.at[pl.ds(s * RPT, RPT)])
    pltpu.sync_copy(dst_hbm.at[wid], dst_v)
    pltpu.sync_copy(ones_hbm, ones_v)
    plsc.subcore_barrier()

    @pl.loop(0, NCHUNK)
    def _chunk(i):
        pltpu.sync_copy(ones_v, acc.at[dst_v.at[i]], add=True)

    plsc.subcore_barrier()
    pltpu.sync_copy(acc.at[pl.ds(s * RPT, RPT)], out_hbm.at[c, pl.ds(s * RPT, RPT)])


@functools.partial(
    pl.kernel,
    out_type=jax.ShapeDtypeStruct((NC, NP, D), jnp.float32),
    mesh=_MESH,
    scratch_types=[
        pltpu.VMEM((NCHUNK, K), jnp.int32),
        pltpu.VMEM((K, D), jnp.float32),
        pltpu.VMEM_SHARED((NP, D), jnp.float32),
        pltpu.SemaphoreType.DMA,
    ],
)
def _sc_deg(dst_hbm, ones_hbm, zeros_hbm, out_hbm, dst_v, ones_v, acc, sem):
    _deg_body(dst_hbm, ones_hbm, zeros_hbm, out_hbm, dst_v, ones_v, acc, sem)


def _agg_body(t_hbm, src_hbm, dst_hbm, zeros_hbm, out_hbm,
              src_v, dst_v, rows_v, acc, sem):
    c = lax.axis_index("c")
    s = lax.axis_index("s")
    wid = c * NS + s

    @pl.when(c == 0)
    def _():
        pltpu.sync_copy(t_hbm.at[pl.ds(s * RPT, RPT)], acc.at[pl.ds(s * RPT, RPT)])

    @pl.when(c != 0)
    def _():
        pltpu.sync_copy(zeros_hbm.at[pl.ds(s * RPT, RPT)], acc.at[pl.ds(s * RPT, RPT)])

    pltpu.sync_copy(src_hbm.at[wid], src_v)
    pltpu.sync_copy(dst_hbm.at[wid], dst_v)
    plsc.subcore_barrier()

    @pl.loop(0, NCHUNK)
    def _chunk(i):
        pltpu.sync_copy(t_hbm.at[src_v.at[i]], rows_v)
        pltpu.sync_copy(rows_v, acc.at[dst_v.at[i]], add=True)

    plsc.subcore_barrier()
    pltpu.sync_copy(acc.at[pl.ds(s * RPT, RPT)], out_hbm.at[c, pl.ds(s * RPT, RPT)])


@functools.partial(
    pl.kernel,
    out_type=jax.ShapeDtypeStruct((NC, NP, D), jnp.float32),
    mesh=_MESH,
    scratch_types=[
        pltpu.VMEM((NCHUNK, K), jnp.int32),
        pltpu.VMEM((NCHUNK, K), jnp.int32),
        pltpu.VMEM((K, D), jnp.float32),
        pltpu.VMEM_SHARED((NP, D), jnp.float32),
        pltpu.SemaphoreType.DMA,
    ],
)
def _sc_agg(t_hbm, src_hbm, dst_hbm, zeros_hbm, out_hbm,
            src_v, dst_v, rows_v, acc, sem):
    _agg_body(t_hbm, src_hbm, dst_hbm, zeros_hbm, out_hbm,
              src_v, dst_v, rows_v, acc, sem)


_BN = 1024


def _tc1_body(deg2_ref, x_ref, w_ref, t_ref, dinv_ref):
    deg = deg2_ref[0, :, 0:1] + deg2_ref[1, :, 0:1] + 1.0
    dinv = lax.rsqrt(deg)
    h = jnp.dot(x_ref[...], w_ref[...], precision=lax.Precision.HIGHEST,
                preferred_element_type=jnp.float32)
    t_ref[...] = dinv * h
    dinv_ref[...] = dinv


def _tc1(deg2, x, w1):
    return pl.pallas_call(
        _tc1_body,
        grid=(NP // _BN,),
        in_specs=[
            pl.BlockSpec((NC, _BN, D), lambda i: (0, i, 0)),
            pl.BlockSpec((_BN, D), lambda i: (i, 0)),
            pl.BlockSpec((D, D), lambda i: (0, 0)),
        ],
        out_specs=[
            pl.BlockSpec((_BN, D), lambda i: (i, 0)),
            pl.BlockSpec((_BN, 1), lambda i: (i, 0)),
        ],
        out_shape=[
            jax.ShapeDtypeStruct((NP, D), jnp.float32),
            jax.ShapeDtypeStruct((NP, 1), jnp.float32),
        ],
    )(deg2, x, w1)


def _tc2_body(agg_ref, dinv_ref, b_ref, w_ref, t_ref):
    g = agg_ref[0] + agg_ref[1]
    z = jnp.maximum(dinv_ref[...] * g + b_ref[...], 0.0)
    h = jnp.dot(z, w_ref[...], precision=lax.Precision.HIGHEST,
                preferred_element_type=jnp.float32)
    t_ref[...] = dinv_ref[...] * h


def _tc2(agg1, dinv, b1, w2):
    return pl.pallas_call(
        _tc2_body,
        grid=(NP // _BN,),
        in_specs=[
            pl.BlockSpec((NC, _BN, D), lambda i: (0, i, 0)),
            pl.BlockSpec((_BN, 1), lambda i: (i, 0)),
            pl.BlockSpec((D,), lambda i: (0,)),
            pl.BlockSpec((D, D), lambda i: (0, 0)),
        ],
        out_specs=pl.BlockSpec((_BN, D), lambda i: (i, 0)),
        out_shape=jax.ShapeDtypeStruct((NP, D), jnp.float32),
    )(agg1, dinv, b1, w2)


def _tc3_body(agg_ref, dinv_ref, b_ref, out_ref):
    out_ref[...] = dinv_ref[...] * (agg_ref[0] + agg_ref[1]) + b_ref[...]


def _tc3(agg2, dinv, b2):
    return pl.pallas_call(
        _tc3_body,
        grid=(NP // _BN,),
        in_specs=[
            pl.BlockSpec((NC, _BN, D), lambda i: (0, i, 0)),
            pl.BlockSpec((_BN, 1), lambda i: (i, 0)),
            pl.BlockSpec((D,), lambda i: (0,)),
        ],
        out_specs=pl.BlockSpec((_BN, D), lambda i: (i, 0)),
        out_shape=jax.ShapeDtypeStruct((NP, D), jnp.float32),
    )(agg2, dinv, b2)


def kernel(x, edge_index, W1, b1, W2, b2):
    src = edge_index[0].astype(jnp.int32).reshape(NW, NCHUNK, K)
    dst = edge_index[1].astype(jnp.int32).reshape(NW, NCHUNK, K)

    xp = jnp.pad(x, ((0, NP - N), (0, 0)))
    ones = jnp.zeros((K, D), jnp.float32).at[:, 0].set(1.0)
    zeros128 = jnp.zeros((NP, D), jnp.float32)

    deg2 = _sc_deg(dst, ones, zeros128)
    t1, dinv = _tc1(deg2, xp, W1)
    agg1 = _sc_agg(t1, src, dst, zeros128)
    t2 = _tc2(agg1, dinv, b1, W2)
    agg2 = _sc_agg(t2, src, dst, zeros128)
    return _tc3(agg2, dinv, b2)[:N]


# speedup vs baseline: 27.9833x; 1.3802x over previous
import functools

import jax
import jax.numpy as jnp
from jax import lax
from jax.experimental import pallas as pl
from jax.experimental.pallas import tpu as pltpu
from jax.experimental.pallas import tpu_sc as plsc

N = 10000
NP = 10240
E = 320000
D = 128

NC = 2
NS = 16
NW = NC * NS
EPW = E // NW
K = 125
NCHUNK = EPW // K
NB = 2
NBLK = NCHUNK // 8
RPT = NP // NS

_MESH = plsc.VectorSubcoreMesh(
    core_axis_name="c", subcore_axis_name="s", num_cores=NC, num_subcores=NS
)


def _deg_body(dst_hbm, ones_hbm, zeros_hbm, out_hbm, dst_v, ones_v, acc, sem):
    c = lax.axis_index("c")
    s = lax.axis_index("s")
    wid = c * NS + s
    pltpu.sync_copy(zeros_hbm.at[pl.ds(s * RPT, RPT)], acc.at[pl.ds(s * RPT, RPT)])
    pltpu.sync_copy(dst_hbm.at[wid], dst_v)
    pltpu.sync_copy(ones_hbm, ones_v)
    plsc.subcore_barrier()

    @pl.loop(0, NCHUNK, step=8)
    def _grp(i0):
        for b in range(8):
            pltpu.async_copy(ones_v, acc.at[dst_v.at[i0 + b]], sem, add=True)
        for b in range(8):
            pltpu.make_async_copy(ones_v, acc.at[pl.ds(0, K)], sem).wait()

    plsc.subcore_barrier()
    pltpu.sync_copy(acc.at[pl.ds(s * RPT, RPT)], out_hbm.at[c, pl.ds(s * RPT, RPT)])


@functools.partial(
    pl.kernel,
    out_type=jax.ShapeDtypeStruct((NC, NP, D), jnp.float32),
    mesh=_MESH,
    scratch_types=[
        pltpu.VMEM((NCHUNK, K), jnp.int32),
        pltpu.VMEM((K, D), jnp.float32),
        pltpu.VMEM_SHARED((NP, D), jnp.float32),
        pltpu.SemaphoreType.DMA,
    ],
)
def _sc_deg(dst_hbm, ones_hbm, zeros_hbm, out_hbm, dst_v, ones_v, acc, sem):
    _deg_body(dst_hbm, ones_hbm, zeros_hbm, out_hbm, dst_v, ones_v, acc, sem)


def _agg_body(t_hbm, src_hbm, dst_hbm, zeros_hbm, out_hbm,
              src_v, dstb, rows, acc, gsem, ssem, dsem):
    c = lax.axis_index("c")
    s = lax.axis_index("s")
    wid = c * NS + s

    @pl.when(c == 0)
    def _():
        pltpu.sync_copy(t_hbm.at[pl.ds(s * RPT, RPT)], acc.at[pl.ds(s * RPT, RPT)])

    @pl.when(c != 0)
    def _():
        pltpu.sync_copy(zeros_hbm.at[pl.ds(s * RPT, RPT)], acc.at[pl.ds(s * RPT, RPT)])

    pltpu.sync_copy(src_hbm.at[wid], src_v)
    pltpu.sync_copy(dst_hbm.at[wid, 0], dstb.at[0])
    plsc.subcore_barrier()

    pltpu.async_copy(t_hbm.at[src_v.at[0]], rows.at[0], gsem)
    pltpu.async_copy(t_hbm.at[src_v.at[1]], rows.at[1], gsem)

    @pl.loop(0, NBLK, step=2)
    def _blk2(j0):
        for jj in range(2):
            j = j0 + jj

            @pl.when(j < NBLK - 1)
            def _():
                pltpu.async_copy(dst_hbm.at[wid, j + 1], dstb.at[1 - jj], dsem)

            for cc in range(8):
                i = j * 8 + cc
                b = cc % 2
                pltpu.make_async_copy(t_hbm.at[src_v.at[i]], rows.at[b], gsem).wait()
                pltpu.async_copy(rows.at[b], acc.at[dstb.at[jj, cc]], ssem, add=True)

                @pl.when(i < NCHUNK - 2)
                def _():
                    pltpu.make_async_copy(rows.at[b], acc.at[pl.ds(0, K)], ssem).wait()
                    pltpu.async_copy(t_hbm.at[src_v.at[i + 2]], rows.at[b], gsem)

            @pl.when(j < NBLK - 1)
            def _():
                pltpu.make_async_copy(dst_hbm.at[wid, 0], dstb.at[1 - jj], dsem).wait()

    for b in range(2):
        pltpu.make_async_copy(rows.at[b], acc.at[pl.ds(0, K)], ssem).wait()

    plsc.subcore_barrier()
    pltpu.sync_copy(acc.at[pl.ds(s * RPT, RPT)], out_hbm.at[c, pl.ds(s * RPT, RPT)])


@functools.partial(
    pl.kernel,
    out_type=jax.ShapeDtypeStruct((NC, NP, D), jnp.float32),
    mesh=_MESH,
    scratch_types=[
        pltpu.VMEM((NCHUNK, K), jnp.int32),
        pltpu.VMEM((2, 8, K), jnp.int32),
        pltpu.VMEM((2, K, D), jnp.float32),
        pltpu.VMEM_SHARED((NP, D), jnp.float32),
        pltpu.SemaphoreType.DMA,
        pltpu.SemaphoreType.DMA,
        pltpu.SemaphoreType.DMA,
    ],
)
def _sc_agg(t_hbm, src_hbm, dst_hbm, zeros_hbm, out_hbm,
            src_v, dstb, rows, acc, gsem, ssem, dsem):
    _agg_body(t_hbm, src_hbm, dst_hbm, zeros_hbm, out_hbm,
              src_v, dstb, rows, acc, gsem, ssem, dsem)


_BN = 1024


def _tc1_body(deg2_ref, x_ref, w_ref, t_ref, dinv_ref):
    deg = deg2_ref[0, :, 0:1] + deg2_ref[1, :, 0:1] + 1.0
    dinv = lax.rsqrt(deg)
    h = jnp.dot(x_ref[...], w_ref[...], precision=lax.Precision.HIGHEST,
                preferred_element_type=jnp.float32)
    t_ref[...] = dinv * h
    dinv_ref[...] = dinv


def _tc1(deg2, x, w1):
    return pl.pallas_call(
        _tc1_body,
        grid=(NP // _BN,),
        in_specs=[
            pl.BlockSpec((NC, _BN, D), lambda i: (0, i, 0)),
            pl.BlockSpec((_BN, D), lambda i: (i, 0)),
            pl.BlockSpec((D, D), lambda i: (0, 0)),
        ],
        out_specs=[
            pl.BlockSpec((_BN, D), lambda i: (i, 0)),
            pl.BlockSpec((_BN, 1), lambda i: (i, 0)),
        ],
        out_shape=[
            jax.ShapeDtypeStruct((NP, D), jnp.float32),
            jax.ShapeDtypeStruct((NP, 1), jnp.float32),
        ],
    )(deg2, x, w1)


def _tc2_body(agg_ref, dinv_ref, b_ref, w_ref, t_ref):
    g = agg_ref[0] + agg_ref[1]
    z = jnp.maximum(dinv_ref[...] * g + b_ref[...], 0.0)
    h = jnp.dot(z, w_ref[...], precision=lax.Precision.HIGHEST,
                preferred_element_type=jnp.float32)
    t_ref[...] = dinv_ref[...] * h


def _tc2(agg1, dinv, b1, w2):
    return pl.pallas_call(
        _tc2_body,
        grid=(NP // _BN,),
        in_specs=[
            pl.BlockSpec((NC, _BN, D), lambda i: (0, i, 0)),
            pl.BlockSpec((_BN, 1), lambda i: (i, 0)),
            pl.BlockSpec((D,), lambda i: (0,)),
            pl.BlockSpec((D, D), lambda i: (0, 0)),
        ],
        out_specs=pl.BlockSpec((_BN, D), lambda i: (i, 0)),
        out_shape=jax.ShapeDtypeStruct((NP, D), jnp.float32),
    )(agg1, dinv, b1, w2)


def _tc3_body(agg_ref, dinv_ref, b_ref, out_ref):
    out_ref[...] = dinv_ref[...] * (agg_ref[0] + agg_ref[1]) + b_ref[...]


def _tc3(agg2, dinv, b2):
    return pl.pallas_call(
        _tc3_body,
        grid=(NP // _BN,),
        in_specs=[
            pl.BlockSpec((NC, _BN, D), lambda i: (0, i, 0)),
            pl.BlockSpec((_BN, 1), lambda i: (i, 0)),
            pl.BlockSpec((D,), lambda i: (0,)),
        ],
        out_specs=pl.BlockSpec((_BN, D), lambda i: (i, 0)),
        out_shape=jax.ShapeDtypeStruct((NP, D), jnp.float32),
    )(agg2, dinv, b2)


def kernel(x, edge_index, W1, b1, W2, b2):
    src = edge_index[0].astype(jnp.int32).reshape(NW, NCHUNK, K)
    dst = edge_index[1].astype(jnp.int32).reshape(NW, NCHUNK, K)

    xp = jnp.pad(x, ((0, NP - N), (0, 0)))
    ones = jnp.zeros((K, D), jnp.float32).at[:, 0].set(1.0)
    zeros128 = jnp.zeros((NP, D), jnp.float32)

    deg2 = _sc_deg(dst, ones, zeros128)
    t1, dinv = _tc1(deg2, xp, W1)
    dst4 = dst.reshape(NW, NBLK, 8, K)
    agg1 = _sc_agg(t1, src, dst4, zeros128)
    t2 = _tc2(agg1, dinv, b1, W2)
    agg2 = _sc_agg(t2, src, dst4, zeros128)
    return _tc3(agg2, dinv, b2)[:N]


# speedup vs baseline: 28.1645x; 1.0065x over previous
import functools

import jax
import jax.numpy as jnp
from jax import lax
from jax.experimental import pallas as pl
from jax.experimental.pallas import tpu as pltpu
from jax.experimental.pallas import tpu_sc as plsc

N = 10000
NP = 10240
E = 320000
D = 128

NC = 2
NS = 16
NW = NC * NS
EPW = E // NW
K = 125
NCHUNK = EPW // K
NB = 2
NBLK = NCHUNK // 8
RPT = NP // NS

_MESH = plsc.VectorSubcoreMesh(
    core_axis_name="c", subcore_axis_name="s", num_cores=NC, num_subcores=NS
)


def _deg_body(dst_hbm, ones_hbm, zeros_hbm, out_hbm, dst_v, ones_v, acc, sem):
    c = lax.axis_index("c")
    s = lax.axis_index("s")
    wid = c * NS + s
    pltpu.sync_copy(zeros_hbm.at[pl.ds(s * RPT, RPT)], acc.at[pl.ds(s * RPT, RPT)])
    pltpu.sync_copy(dst_hbm.at[wid], dst_v)
    pltpu.sync_copy(ones_hbm, ones_v)
    plsc.subcore_barrier()

    @pl.loop(0, NCHUNK, step=8)
    def _grp(i0):
        for b in range(8):
            pltpu.async_copy(ones_v, acc.at[dst_v.at[i0 + b]], sem, add=True)
        for b in range(8):
            pltpu.make_async_copy(ones_v, acc.at[pl.ds(0, K)], sem).wait()

    plsc.subcore_barrier()
    pltpu.sync_copy(acc.at[pl.ds(s * RPT, RPT)], out_hbm.at[c, pl.ds(s * RPT, RPT)])


@functools.partial(
    pl.kernel,
    out_type=jax.ShapeDtypeStruct((NC, NP, D), jnp.float32),
    mesh=_MESH,
    scratch_types=[
        pltpu.VMEM((NCHUNK, K), jnp.int32),
        pltpu.VMEM((K, D), jnp.float32),
        pltpu.VMEM_SHARED((NP, D), jnp.float32),
        pltpu.SemaphoreType.DMA,
    ],
)
def _sc_deg(dst_hbm, ones_hbm, zeros_hbm, out_hbm, dst_v, ones_v, acc, sem):
    _deg_body(dst_hbm, ones_hbm, zeros_hbm, out_hbm, dst_v, ones_v, acc, sem)


def _agg_body(t_hbm, src_hbm, dst_hbm, zeros_hbm, out_hbm,
              src_v, dstb, rows, acc, gsem, ssem, dsem):
    c = lax.axis_index("c")
    s = lax.axis_index("s")
    wid = c * NS + s

    @pl.when(c == 0)
    def _():
        pltpu.sync_copy(t_hbm.at[pl.ds(s * RPT, RPT)], acc.at[pl.ds(s * RPT, RPT)])

    @pl.when(c != 0)
    def _():
        pltpu.sync_copy(zeros_hbm.at[pl.ds(s * RPT, RPT)], acc.at[pl.ds(s * RPT, RPT)])

    pltpu.sync_copy(src_hbm.at[wid], src_v)
    pltpu.sync_copy(dst_hbm.at[wid, 0], dstb.at[0])
    plsc.subcore_barrier()

    pltpu.async_copy(t_hbm.at[src_v.at[0]], rows.at[0], gsem)
    pltpu.async_copy(t_hbm.at[src_v.at[1]], rows.at[1], gsem)

    @pl.loop(0, NBLK, step=2)
    def _blk2(j0):
        for jj in range(2):
            j = j0 + jj

            @pl.when(j < NBLK - 1)
            def _():
                pltpu.async_copy(dst_hbm.at[wid, j + 1], dstb.at[1 - jj], dsem)

            for cc in range(8):
                i = j * 8 + cc
                b = cc % 2
                pltpu.make_async_copy(t_hbm.at[src_v.at[i]], rows.at[b], gsem).wait()
                pltpu.async_copy(rows.at[b], acc.at[dstb.at[jj, cc]], ssem, add=True)

                @pl.when(i < NCHUNK - 2)
                def _():
                    pltpu.make_async_copy(rows.at[b], acc.at[pl.ds(0, K)], ssem).wait()
                    pltpu.async_copy(t_hbm.at[src_v.at[i + 2]], rows.at[b], gsem)

            @pl.when(j < NBLK - 1)
            def _():
                pltpu.make_async_copy(dst_hbm.at[wid, 0], dstb.at[1 - jj], dsem).wait()

    for b in range(2):
        pltpu.make_async_copy(rows.at[b], acc.at[pl.ds(0, K)], ssem).wait()

    plsc.subcore_barrier()
    pltpu.sync_copy(acc.at[pl.ds(s * RPT, RPT)], out_hbm.at[c, pl.ds(s * RPT, RPT)])


@functools.partial(
    pl.kernel,
    out_type=jax.ShapeDtypeStruct((NC, NP, D), jnp.float32),
    mesh=_MESH,
    scratch_types=[
        pltpu.VMEM((NCHUNK, K), jnp.int32),
        pltpu.VMEM((2, 8, K), jnp.int32),
        pltpu.VMEM((2, K, D), jnp.float32),
        pltpu.VMEM_SHARED((NP, D), jnp.float32),
        pltpu.SemaphoreType.DMA,
        pltpu.SemaphoreType.DMA,
        pltpu.SemaphoreType.DMA,
    ],
)
def _sc_agg(t_hbm, src_hbm, dst_hbm, zeros_hbm, out_hbm,
            src_v, dstb, rows, acc, gsem, ssem, dsem):
    _agg_body(t_hbm, src_hbm, dst_hbm, zeros_hbm, out_hbm,
              src_v, dstb, rows, acc, gsem, ssem, dsem)


_BN = 1024


def _mm_body(x_ref, w_ref, h_ref):
    h_ref[...] = jnp.dot(x_ref[...], w_ref[...],
                         precision=lax.Precision.HIGHEST,
                         preferred_element_type=jnp.float32)


def _mm(x, w1):
    return pl.pallas_call(
        _mm_body,
        grid=(NP // _BN,),
        in_specs=[
            pl.BlockSpec((_BN, D), lambda i: (i, 0)),
            pl.BlockSpec((D, D), lambda i: (0, 0)),
        ],
        out_specs=pl.BlockSpec((_BN, D), lambda i: (i, 0)),
        out_shape=jax.ShapeDtypeStruct((NP, D), jnp.float32),
    )(x, w1)


def _tc1_body(deg2_ref, h_ref, t_ref, dinv_ref):
    deg = deg2_ref[0, :, 0:1] + deg2_ref[1, :, 0:1] + 1.0
    dinv = lax.rsqrt(deg)
    t_ref[...] = dinv * h_ref[...]
    dinv_ref[...] = dinv


def _tc1(deg2, h1):
    return pl.pallas_call(
        _tc1_body,
        grid=(NP // _BN,),
        in_specs=[
            pl.BlockSpec((NC, _BN, D), lambda i: (0, i, 0)),
            pl.BlockSpec((_BN, D), lambda i: (i, 0)),
        ],
        out_specs=[
            pl.BlockSpec((_BN, D), lambda i: (i, 0)),
            pl.BlockSpec((_BN, 1), lambda i: (i, 0)),
        ],
        out_shape=[
            jax.ShapeDtypeStruct((NP, D), jnp.float32),
            jax.ShapeDtypeStruct((NP, 1), jnp.float32),
        ],
    )(deg2, h1)


def _tc2_body(agg_ref, dinv_ref, b_ref, w_ref, t_ref):
    g = agg_ref[0] + agg_ref[1]
    z = jnp.maximum(dinv_ref[...] * g + b_ref[...], 0.0)
    h = jnp.dot(z, w_ref[...], precision=lax.Precision.HIGHEST,
                preferred_element_type=jnp.float32)
    t_ref[...] = dinv_ref[...] * h


def _tc2(agg1, dinv, b1, w2):
    return pl.pallas_call(
        _tc2_body,
        grid=(NP // _BN,),
        in_specs=[
            pl.BlockSpec((NC, _BN, D), lambda i: (0, i, 0)),
            pl.BlockSpec((_BN, 1), lambda i: (i, 0)),
            pl.BlockSpec((D,), lambda i: (0,)),
            pl.BlockSpec((D, D), lambda i: (0, 0)),
        ],
        out_specs=pl.BlockSpec((_BN, D), lambda i: (i, 0)),
        out_shape=jax.ShapeDtypeStruct((NP, D), jnp.float32),
    )(agg1, dinv, b1, w2)


def _tc3_body(agg_ref, dinv_ref, b_ref, out_ref):
    out_ref[...] = dinv_ref[...] * (agg_ref[0] + agg_ref[1]) + b_ref[...]


def _tc3(agg2, dinv, b2):
    return pl.pallas_call(
        _tc3_body,
        grid=(NP // _BN,),
        in_specs=[
            pl.BlockSpec((NC, _BN, D), lambda i: (0, i, 0)),
            pl.BlockSpec((_BN, 1), lambda i: (i, 0)),
            pl.BlockSpec((D,), lambda i: (0,)),
        ],
        out_specs=pl.BlockSpec((_BN, D), lambda i: (i, 0)),
        out_shape=jax.ShapeDtypeStruct((NP, D), jnp.float32),
    )(agg2, dinv, b2)


def kernel(x, edge_index, W1, b1, W2, b2):
    src = edge_index[0].astype(jnp.int32).reshape(NW, NCHUNK, K)
    dst = edge_index[1].astype(jnp.int32).reshape(NW, NCHUNK, K)

    xp = jnp.pad(x, ((0, NP - N), (0, 0)))
    ones = jnp.zeros((K, D), jnp.float32).at[:, 0].set(1.0)
    zeros128 = jnp.zeros((NP, D), jnp.float32)

    deg2 = _sc_deg(dst, ones, zeros128)
    h1 = _mm(xp, W1)
    t1, dinv = _tc1(deg2, h1)
    dst4 = dst.reshape(NW, NBLK, 8, K)
    agg1 = _sc_agg(t1, src, dst4, zeros128)
    t2 = _tc2(agg1, dinv, b1, W2)
    agg2 = _sc_agg(t2, src, dst4, zeros128)
    return _tc3(agg2, dinv, b2)[:N]
